# baseline (speedup 1.0000x reference)
import functools
import math

import jax
import jax.numpy as jnp
from jax import lax
from jax.experimental import pallas as pl
from jax.experimental.pallas import tpu as pltpu
from jax.experimental.pallas import tpu_sc as plsc

S = 64.0
B = 1024
C = 100000
NTILE = 781
CT = NTILE * 128
NBAND = B // 8
LOG_TERM = math.log(C - 1) / S

_info = plsc.get_sparse_core_info()
_NC, _NS = _info.num_cores, _info.num_subcores
NW = _NC * _NS
BPW = NBAND // NW
TPW = B // NW

_CHT = 32
_CHC = _CHT * 128
_NFULL = NTILE // _CHT
_TAILC = (NTILE - _NFULL * _CHT) * 128

_mesh = plsc.VectorSubcoreMesh(core_axis_name="c", subcore_axis_name="s")
_params = pltpu.CompilerParams(use_tc_tiling_on_sc=True)


@functools.partial(
    pl.kernel,
    out_type=jax.ShapeDtypeStruct((NW, 16), jnp.float32),
    mesh=_mesh,
    scratch_types=[
        pltpu.VMEM((8, _CHC), jnp.float32),
        pltpu.VMEM((16,), jnp.float32),
        pltpu.SemaphoreType.DMA,
    ],
    compiler_params=_params,
)
def _sc_scale(out_ref, logits_hbm, psum_hbm, buf, accv, sem):
    wid = lax.axis_index("s") * _NC + lax.axis_index("c")

    def do_chunk(r0, c0, ncols, acc):
        src = logits_hbm.at[pl.ds(r0, 8), pl.ds(c0, ncols)]
        dst = out_ref.at[pl.ds(r0, 8), pl.ds(c0, ncols)]
        bslice = buf.at[pl.ds(0, 8), pl.ds(0, ncols)]
        pltpu.sync_copy(src, bslice)

        def col_body(k, a):
            for s in range(8):
                sl = pl.ds(k * 16, 16)
                v = buf[s, sl]
                a = a + v
                buf[s, sl] = v * S
            return a

        acc = lax.fori_loop(0, ncols // 16, col_body, acc)
        pltpu.sync_copy(bslice, dst)
        return acc

    def band_body(bi, acc):
        r0 = pl.multiple_of((wid * BPW + bi) * 8, 8)

        def chunk_body(ch, a):
            c0 = pl.multiple_of(ch * _CHC, 128)
            return do_chunk(r0, c0, _CHC, a)

        acc = lax.fori_loop(0, _NFULL, chunk_body, acc)
        acc = do_chunk(r0, _NFULL * _CHC, _TAILC, acc)
        return acc

    acc = lax.fori_loop(0, BPW, band_body, jnp.zeros((16,), jnp.float32))
    accv[...] = acc
    pltpu.sync_copy(accv, psum_hbm.at[wid])


@functools.partial(
    pl.kernel,
    out_type=jax.ShapeDtypeStruct((NW, 16), jnp.float32),
    mesh=_mesh,
    scratch_types=[
        pltpu.VMEM((8, 128), jnp.float32),
        pltpu.VMEM((TPW,), jnp.int32),
        pltpu.VMEM((TPW,), jnp.int32),
        pltpu.VMEM((16,), jnp.float32),
        pltpu.SemaphoreType.DMA,
    ],
    compiler_params=_params,
)
def _sc_tgt(logits_hbm, ct_hbm, lp_hbm, tpart_hbm, tile, ct_v, lp_v, accv, sem):
    wid = lax.axis_index("s") * _NC + lax.axis_index("c")
    base = wid * TPW
    pltpu.sync_copy(ct_hbm.at[pl.ds(base, TPW)], ct_v)
    pltpu.sync_copy(lp_hbm.at[pl.ds(base, TPW)], lp_v)

    lane = lax.iota(jnp.int32, 16)
    zero = jnp.zeros((16,), jnp.float32)
    accv[...] = zero
    for t in range(TPW):
        row = base + t
        ctv = ct_v[pl.ds((t // 16) * 16, 16)]
        lpv = lp_v[pl.ds((t // 16) * 16, 16)]
        ct = ctv[t % 16]
        lp = lpv[t % 16]

        @pl.when(ct < NTILE)
        def _():
            c0 = pl.multiple_of(ct * 128, 128)
            r0 = (row // 8) * 8
            pltpu.sync_copy(logits_hbm.at[pl.ds(r0, 8), pl.ds(c0, 128)], tile)
            aa = pl.multiple_of((lp // 16) * 16, 16)
            v = tile[row % 8, pl.ds(aa, 16)]
            accv[...] = accv[...] + jnp.where(lane == lp - aa, v, zero)

    pltpu.sync_copy(accv, tpart_hbm.at[wid])


@functools.partial(
    pl.kernel,
    out_type=jax.ShapeDtypeStruct((16,), jnp.float32),
    mesh=_mesh,
    scratch_types=[
        pltpu.VMEM((8, 128), jnp.float32),
        pltpu.VMEM((TPW,), jnp.int32),
        pltpu.VMEM((TPW,), jnp.int32),
        pltpu.VMEM((NW, 16), jnp.float32),
        pltpu.VMEM((NW, 16), jnp.float32),
        pltpu.VMEM((2, 16), jnp.float32),
        pltpu.VMEM((16,), jnp.float32),
        pltpu.SemaphoreType.DMA,
    ],
    compiler_params=_params,
)
def _sc_fix(out_ref, ct_hbm, lp_hbm, psum_hbm, tpart_hbm, extras_hbm,
            corr_hbm, tile, ct_v, lp_v, ps_v, tp_v, ex_v, cv, sem):
    wid = lax.axis_index("s") * _NC + lax.axis_index("c")
    base = wid * TPW
    pltpu.sync_copy(ct_hbm.at[pl.ds(base, TPW)], ct_v)
    pltpu.sync_copy(lp_hbm.at[pl.ds(base, TPW)], lp_v)
    pltpu.sync_copy(psum_hbm, ps_v)
    pltpu.sync_copy(tpart_hbm, tp_v)
    pltpu.sync_copy(extras_hbm, ex_v)

    sa = ex_v[0]
    st = ex_v[1]
    for w in range(NW):
        sa = sa + ps_v[w]
        st = st + tp_v[w]
    lane = lax.iota(jnp.int32, 16)
    for sh in (1, 2, 4, 8):
        sa = sa + sa.at[lane ^ sh].get(mode="promise_in_bounds")
        st = st + st.at[lane ^ sh].get(mode="promise_in_bounds")

    avg_p = st * (1.0 / B)
    avg_n = (sa - st) * (1.0 / (B * (C - 1)))
    corr = (avg_p - avg_n - LOG_TERM) * S

    @pl.when(wid == 0)
    def _():
        cv[...] = corr
        pltpu.sync_copy(cv, corr_hbm)

    zero = jnp.zeros((16,), jnp.float32)
    for t in range(TPW):
        row = base + t
        ctv = ct_v[pl.ds((t // 16) * 16, 16)]
        lpv = lp_v[pl.ds((t // 16) * 16, 16)]
        ct = ctv[t % 16]
        lp = lpv[t % 16]

        @pl.when(ct < NTILE)
        def _():
            c0 = pl.multiple_of(ct * 128, 128)
            r0 = (row // 8) * 8
            win = out_ref.at[pl.ds(r0, 8), pl.ds(c0, 128)]
            pltpu.sync_copy(win, tile)
            aa = pl.multiple_of((lp // 16) * 16, 16)
            v = tile[row % 8, pl.ds(aa, 16)]
            tile[row % 8, pl.ds(aa, 16)] = jnp.where(
                lane == lp - aa, v - corr, v)
            pltpu.sync_copy(tile, win)


def kernel(logits, labels):
    labels = labels.astype(jnp.int32)
    ct = labels // 128
    lp = labels % 128

    out_ref = jax.empty_ref(jax.ShapeDtypeStruct((B, C), jnp.float32))
    psum = _sc_scale(out_ref, logits)
    tpart = _sc_tgt(logits, ct, lp)

    tail = logits[:, CT:]
    cols = jnp.arange(C - CT, dtype=jnp.int32)[None, :] + CT
    mask2d = labels[:, None] == cols
    extras = jnp.stack([
        jnp.full((16,), jnp.sum(tail) / 16.0, jnp.float32),
        jnp.full((16,), jnp.sum(jnp.where(mask2d, tail, 0.0)) / 16.0,
                 jnp.float32),
    ])
    corr = _sc_fix(out_ref, ct, lp, psum, tpart, extras)

    tail_final = jnp.where(mask2d, tail * S - corr[0], tail * S)
    out_ref[:, CT:] = tail_final
    return out_ref[...]


# speedup vs baseline: 1.0029x; 1.0029x over previous
import functools
import math

import jax
import jax.numpy as jnp
from jax import lax
from jax.experimental import pallas as pl
from jax.experimental.pallas import tpu as pltpu
from jax.experimental.pallas import tpu_sc as plsc

S = 64.0
B = 1024
C = 100000
NTILE = 781
CT = NTILE * 128
NBAND = B // 8
LOG_TERM = math.log(C - 1) / S

_info = plsc.get_sparse_core_info()
_NC, _NS = _info.num_cores, _info.num_subcores
NW = _NC * _NS
BPW = NBAND // NW
TPW = B // NW

_CHT = 32
_CHC = _CHT * 128
_NFULL = NTILE // _CHT
_TAILC = (NTILE - _NFULL * _CHT) * 128

_mesh = plsc.VectorSubcoreMesh(core_axis_name="c", subcore_axis_name="s")
_params = pltpu.CompilerParams(use_tc_tiling_on_sc=True)


@functools.partial(
    pl.kernel,
    out_type=(
        jax.ShapeDtypeStruct((NW, 16), jnp.float32),
        jax.ShapeDtypeStruct((NW, 16), jnp.float32),
    ),
    mesh=_mesh,
    scratch_types=[
        pltpu.VMEM((8, _CHC), jnp.float32),
        pltpu.VMEM((16,), jnp.float32),
        pltpu.VMEM((8, 128), jnp.float32),
        pltpu.VMEM((TPW,), jnp.int32),
        pltpu.VMEM((TPW,), jnp.int32),
        pltpu.VMEM((16,), jnp.float32),
        pltpu.SemaphoreType.DMA,
    ],
    compiler_params=_params,
)
def _sc_scale(out_ref, logits_hbm, ct_hbm, lp_hbm, psum_hbm, tpart_hbm,
              buf, accv, tile, ct_v, lp_v, taccv, sem):
    wid = lax.axis_index("s") * _NC + lax.axis_index("c")

    def do_chunk(r0, c0, ncols, acc):
        src = logits_hbm.at[pl.ds(r0, 8), pl.ds(c0, ncols)]
        dst = out_ref.at[pl.ds(r0, 8), pl.ds(c0, ncols)]
        bslice = buf.at[pl.ds(0, 8), pl.ds(0, ncols)]
        pltpu.sync_copy(src, bslice)

        def col_body(k, a):
            for s in range(8):
                sl = pl.ds(k * 16, 16)
                v = buf[s, sl]
                a = a + v
                buf[s, sl] = v * S
            return a

        acc = lax.fori_loop(0, ncols // 16, col_body, acc)
        pltpu.sync_copy(bslice, dst)
        return acc

    def band_body(bi, acc):
        r0 = pl.multiple_of((wid * BPW + bi) * 8, 8)

        def chunk_body(ch, a):
            c0 = pl.multiple_of(ch * _CHC, 128)
            return do_chunk(r0, c0, _CHC, a)

        acc = lax.fori_loop(0, _NFULL, chunk_body, acc)
        acc = do_chunk(r0, _NFULL * _CHC, _TAILC, acc)
        return acc

    acc = lax.fori_loop(0, BPW, band_body, jnp.zeros((16,), jnp.float32))
    accv[...] = acc
    pltpu.sync_copy(accv, psum_hbm.at[wid])

    base = wid * TPW
    pltpu.sync_copy(ct_hbm.at[pl.ds(base, TPW)], ct_v)
    pltpu.sync_copy(lp_hbm.at[pl.ds(base, TPW)], lp_v)
    lane = lax.iota(jnp.int32, 16)
    zero = jnp.zeros((16,), jnp.float32)
    taccv[...] = zero
    for t in range(TPW):
        row = base + t
        ctv = ct_v[pl.ds((t // 16) * 16, 16)]
        lpv = lp_v[pl.ds((t // 16) * 16, 16)]
        tct = ctv[t % 16]
        tlp = lpv[t % 16]

        @pl.when(tct < NTILE)
        def _():
            c0 = pl.multiple_of(tct * 128, 128)
            r0 = (row // 8) * 8
            pltpu.sync_copy(logits_hbm.at[pl.ds(r0, 8), pl.ds(c0, 128)], tile)
            aa = pl.multiple_of((tlp // 16) * 16, 16)
            v = tile[row % 8, pl.ds(aa, 16)]
            taccv[...] = taccv[...] + jnp.where(lane == tlp - aa, v, zero)

    pltpu.sync_copy(taccv, tpart_hbm.at[wid])


@functools.partial(
    pl.kernel,
    out_type=jax.ShapeDtypeStruct((16,), jnp.float32),
    mesh=_mesh,
    scratch_types=[
        pltpu.VMEM((8, 128), jnp.float32),
        pltpu.VMEM((TPW,), jnp.int32),
        pltpu.VMEM((TPW,), jnp.int32),
        pltpu.VMEM((NW, 16), jnp.float32),
        pltpu.VMEM((NW, 16), jnp.float32),
        pltpu.VMEM((2, 16), jnp.float32),
        pltpu.VMEM((16,), jnp.float32),
        pltpu.SemaphoreType.DMA,
    ],
    compiler_params=_params,
)
def _sc_fix(out_ref, ct_hbm, lp_hbm, psum_hbm, tpart_hbm, extras_hbm,
            corr_hbm, tile, ct_v, lp_v, ps_v, tp_v, ex_v, cv, sem):
    wid = lax.axis_index("s") * _NC + lax.axis_index("c")
    base = wid * TPW
    pltpu.sync_copy(ct_hbm.at[pl.ds(base, TPW)], ct_v)
    pltpu.sync_copy(lp_hbm.at[pl.ds(base, TPW)], lp_v)
    pltpu.sync_copy(psum_hbm, ps_v)
    pltpu.sync_copy(tpart_hbm, tp_v)
    pltpu.sync_copy(extras_hbm, ex_v)

    sa = ex_v[0]
    st = ex_v[1]
    for w in range(NW):
        sa = sa + ps_v[w]
        st = st + tp_v[w]
    lane = lax.iota(jnp.int32, 16)
    for sh in (1, 2, 4, 8):
        sa = sa + sa.at[lane ^ sh].get(mode="promise_in_bounds")
        st = st + st.at[lane ^ sh].get(mode="promise_in_bounds")

    avg_p = st * (1.0 / B)
    avg_n = (sa - st) * (1.0 / (B * (C - 1)))
    corr = (avg_p - avg_n - LOG_TERM) * S

    @pl.when(wid == 0)
    def _():
        cv[...] = corr
        pltpu.sync_copy(cv, corr_hbm)

    zero = jnp.zeros((16,), jnp.float32)
    for t in range(TPW):
        row = base + t
        ctv = ct_v[pl.ds((t // 16) * 16, 16)]
        lpv = lp_v[pl.ds((t // 16) * 16, 16)]
        ct = ctv[t % 16]
        lp = lpv[t % 16]

        @pl.when(ct < NTILE)
        def _():
            c0 = pl.multiple_of(ct * 128, 128)
            r0 = (row // 8) * 8
            win = out_ref.at[pl.ds(r0, 8), pl.ds(c0, 128)]
            pltpu.sync_copy(win, tile)
            aa = pl.multiple_of((lp // 16) * 16, 16)
            v = tile[row % 8, pl.ds(aa, 16)]
            tile[row % 8, pl.ds(aa, 16)] = jnp.where(
                lane == lp - aa, v - corr, v)
            pltpu.sync_copy(tile, win)


def kernel(logits, labels):
    labels = labels.astype(jnp.int32)
    ct = labels // 128
    lp = labels % 128

    out_ref = jax.empty_ref(jax.ShapeDtypeStruct((B, C), jnp.float32))
    psum, tpart = _sc_scale(out_ref, logits, ct, lp)

    tail = logits[:, CT:]
    cols = jnp.arange(C - CT, dtype=jnp.int32)[None, :] + CT
    mask2d = labels[:, None] == cols
    extras = jnp.stack([
        jnp.full((16,), jnp.sum(tail) / 16.0, jnp.float32),
        jnp.full((16,), jnp.sum(jnp.where(mask2d, tail, 0.0)) / 16.0,
                 jnp.float32),
    ])
    corr = _sc_fix(out_ref, ct, lp, psum, tpart, extras)

    tail_final = jnp.where(mask2d, tail * S - corr[0], tail * S)
    out_ref[:, CT:] = tail_final
    return jax.freeze(out_ref)
